# baseline (speedup 1.0000x reference)
import functools
import jax
import jax.numpy as jnp
from jax import lax
from jax.experimental import pallas as pl
from jax.experimental.pallas import tpu as pltpu
from jax.experimental.pallas import tpu_sc as plsc

_NC = 2
_NS = 16
_LANES = 16
_MINOR = 128


def kernel(x, pe):
    B, L, D = x.shape
    NW = _NC * _NS
    rows_total = B * L * D // _MINOR
    rows_per_w = rows_total // NW
    rows_per_l = L * D // _MINOR
    CH = 384
    n_chunks = rows_per_w // CH

    xf = x.reshape(rows_total, _MINOR)
    pef = pe[:L].reshape(rows_per_l, _MINOR)

    mesh = plsc.VectorSubcoreMesh(core_axis_name="c", subcore_axis_name="s")

    @functools.partial(
        pl.kernel,
        out_type=jax.ShapeDtypeStruct((rows_total, _MINOR), jnp.float32),
        mesh=mesh,
        scratch_types=[
            pltpu.VMEM((CH, _MINOR), jnp.float32),
            pltpu.VMEM((CH, _MINOR), jnp.float32),
        ],
    )
    def sc_add(x_hbm, pe_hbm, out_hbm, xbuf, pebuf):
        wid = lax.axis_index("s") * _NC + lax.axis_index("c")
        base = wid * rows_per_w
        pe_base = base % rows_per_l

        def chunk_body(ci, _):
            r0 = base + ci * CH
            p0 = pe_base + ci * CH
            pltpu.sync_copy(x_hbm.at[pl.ds(r0, CH)], xbuf)
            pltpu.sync_copy(pe_hbm.at[pl.ds(p0, CH)], pebuf)

            def add_body(i, _):
                for j in range(_MINOR // _LANES):
                    sl = pl.ds(j * _LANES, _LANES)
                    xbuf[i, sl] = xbuf[i, sl] + pebuf[i, sl]
                return 0

            lax.fori_loop(0, CH, add_body, 0)
            pltpu.sync_copy(xbuf, out_hbm.at[pl.ds(r0, CH)])
            return 0

        lax.fori_loop(0, n_chunks, chunk_body, 0)

    out = sc_add(xf, pef)
    return out.reshape(B, L, D)


# speedup vs baseline: 1.2727x; 1.2727x over previous
import functools
import jax
import jax.numpy as jnp
from jax import lax
from jax.experimental import pallas as pl
from jax.experimental.pallas import tpu as pltpu
from jax.experimental.pallas import tpu_sc as plsc

_NC = 2
_NS = 16
_LANES = 16
_MINOR = 128
_CH = 128


def kernel(x, pe):
    B, L, D = x.shape
    NW = _NC * _NS
    rows_total = B * L * D // _MINOR
    rows_per_l = L * D // _MINOR
    seg = rows_per_l // NW
    n_chunks = seg // _CH
    n_t = n_chunks * B

    xf = x.reshape(rows_total, _MINOR)
    pef = pe[:L].reshape(rows_per_l, _MINOR)

    mesh = plsc.VectorSubcoreMesh(core_axis_name="c", subcore_axis_name="s")

    @functools.partial(
        pl.kernel,
        out_type=jax.ShapeDtypeStruct((rows_total, _MINOR), jnp.float32),
        mesh=mesh,
        scratch_types=[
            pltpu.VMEM((2, _CH, _MINOR), jnp.float32),
            pltpu.VMEM((2, _CH, _MINOR), jnp.float32),
            pltpu.VMEM((2, _CH, _MINOR), jnp.float32),
            pltpu.SemaphoreType.DMA,
            pltpu.SemaphoreType.DMA,
            pltpu.SemaphoreType.DMA,
            pltpu.SemaphoreType.DMA,
            pltpu.SemaphoreType.DMA,
            pltpu.SemaphoreType.DMA,
        ],
    )
    def sc_add(x_hbm, pe_hbm, out_hbm, xbuf, pebuf, obuf,
               xs0, xs1, ps0, ps1, os0, os1):
        wid = lax.axis_index("s") * _NC + lax.axis_index("c")
        pe0 = wid * seg
        xsem = (xs0, xs1)
        psem = (ps0, ps1)
        osem = (os0, os1)

        def x_row(t):
            ci, b = divmod(t, B)
            return b * rows_per_l + pe0 + ci * _CH

        def start_xload(t):
            return pltpu.async_copy(
                x_hbm.at[pl.ds(x_row(t), _CH)], xbuf.at[t % 2], xsem[t % 2])

        def start_peload(ci):
            return pltpu.async_copy(
                pe_hbm.at[pl.ds(pe0 + ci * _CH, _CH)],
                pebuf.at[ci % 2], psem[ci % 2])

        xcopies = {0: start_xload(0), 1: start_xload(1)}
        pecopies = {0: start_peload(0)}
        if n_chunks > 1:
            pecopies[1] = start_peload(1)
        ocopies = {}

        for t in range(n_t):
            ci, b = divmod(t, B)
            sl = t % 2
            xcopies.pop(t).wait()
            if b == 0 and ci in pecopies:
                pecopies.pop(ci).wait()
            if t >= 2:
                ocopies.pop(t - 2).wait()

            def add_body(i, _):
                for j in range(_MINOR // _LANES):
                    s = pl.ds(j * _LANES, _LANES)
                    obuf[sl, i, s] = xbuf[sl, i, s] + pebuf[ci % 2, i, s]
                return 0

            lax.fori_loop(0, _CH, add_body, 0)

            ocopies[t] = pltpu.async_copy(
                obuf.at[sl], out_hbm.at[pl.ds(x_row(t), _CH)], osem[sl])
            if t + 2 < n_t:
                xcopies[t + 2] = start_xload(t + 2)
            if b == B - 1 and ci + 2 < n_chunks:
                pecopies[ci + 2] = start_peload(ci + 2)

        ocopies.pop(n_t - 2).wait()
        ocopies.pop(n_t - 1).wait()

    out = sc_add(xf, pef)
    return out.reshape(B, L, D)


# speedup vs baseline: 1.2878x; 1.0119x over previous
import functools
import jax
import jax.numpy as jnp
from jax import lax
from jax.experimental import pallas as pl
from jax.experimental.pallas import tpu as pltpu
from jax.experimental.pallas import tpu_sc as plsc

_NC = 2
_NS = 16
_LANES = 16
_MINOR = 128
_CH = 48


def kernel(x, pe):
    B, L, D = x.shape
    NW = _NC * _NS
    rows_total = B * L * D // _MINOR
    rows_per_l = L * D // _MINOR
    seg = rows_per_l // NW
    n_chunks = seg // _CH

    xf = x.reshape(rows_total, _MINOR)
    pef = pe[:L].reshape(rows_per_l, _MINOR)

    mesh = plsc.VectorSubcoreMesh(core_axis_name="c", subcore_axis_name="s")

    @functools.partial(
        pl.kernel,
        out_type=jax.ShapeDtypeStruct((rows_total, _MINOR), jnp.float32),
        mesh=mesh,
        scratch_types=[
            pltpu.VMEM((2, B, _CH, _MINOR), jnp.float32),
            pltpu.VMEM((2, _CH, _MINOR), jnp.float32),
            pltpu.VMEM((2, B, _CH, _MINOR), jnp.float32),
            pltpu.SemaphoreType.DMA,
            pltpu.SemaphoreType.DMA,
            pltpu.SemaphoreType.DMA,
            pltpu.SemaphoreType.DMA,
            pltpu.SemaphoreType.DMA,
            pltpu.SemaphoreType.DMA,
        ],
    )
    def sc_add(x_hbm, pe_hbm, out_hbm, xbuf, pebuf, obuf,
               xs0, xs1, ps0, ps1, os0, os1):
        wid = lax.axis_index("s") * _NC + lax.axis_index("c")
        pe0 = wid * seg
        xsem = (xs0, xs1)
        psem = (ps0, ps1)
        osem = (os0, os1)

        def x_row(ci, b):
            return b * rows_per_l + pe0 + ci * _CH

        def start_loads(ci):
            sl = ci % 2
            xs = [pltpu.async_copy(x_hbm.at[pl.ds(x_row(ci, b), _CH)],
                                   xbuf.at[sl, b], xsem[sl])
                  for b in range(B)]
            ps = pltpu.async_copy(pe_hbm.at[pl.ds(pe0 + ci * _CH, _CH)],
                                  pebuf.at[sl], psem[sl])
            return xs + [ps]

        loads = {0: start_loads(0)}
        if n_chunks > 1:
            loads[1] = start_loads(1)
        stores = {}

        for ci in range(n_chunks):
            sl = ci % 2
            for cp in loads.pop(ci):
                cp.wait()
            if ci >= 2:
                for cp in stores.pop(ci - 2):
                    cp.wait()

            @plsc.parallel_loop(0, _CH)
            def _(i):
                for j in range(_MINOR // _LANES):
                    s = pl.ds(j * _LANES, _LANES)
                    pv = pebuf[sl, i, s]
                    for b in range(B):
                        obuf[sl, b, i, s] = xbuf[sl, b, i, s] + pv

            stores[ci] = [
                pltpu.async_copy(obuf.at[sl, b],
                                 out_hbm.at[pl.ds(x_row(ci, b), _CH)],
                                 osem[sl])
                for b in range(B)]
            if ci + 2 < n_chunks:
                loads[ci + 2] = start_loads(ci + 2)

        for ci in sorted(stores):
            for cp in stores.pop(ci):
                cp.wait()

    out = sc_add(xf, pef)
    return out.reshape(B, L, D)


# speedup vs baseline: 2.4730x; 1.9203x over previous
import functools
import jax
import jax.numpy as jnp
from jax import lax
from jax.experimental import pallas as pl
from jax.experimental.pallas import tpu as pltpu
from jax.experimental.pallas import tpu_sc as plsc

_NC = 2
_NS = 16
_LANES = 16
_CHR = 8


def kernel(x, pe):
    B, L, D = x.shape
    NW = _NC * _NS
    seg = L // NW
    n_chunks = seg // _CHR
    n_half = n_chunks // 2

    xf = x.reshape(B * L, D)
    pef = pe[:L]

    mesh = plsc.VectorSubcoreMesh(core_axis_name="c", subcore_axis_name="s")

    @functools.partial(
        pl.kernel,
        out_type=jax.ShapeDtypeStruct((B * L, D), jnp.float32),
        mesh=mesh,
        scratch_types=[
            pltpu.VMEM((2, B, _CHR, D), jnp.float32),
            pltpu.VMEM((2, _CHR, D), jnp.float32),
            pltpu.VMEM((2, B, _CHR, D), jnp.float32),
            pltpu.SemaphoreType.DMA,
            pltpu.SemaphoreType.DMA,
            pltpu.SemaphoreType.DMA,
            pltpu.SemaphoreType.DMA,
            pltpu.SemaphoreType.DMA,
            pltpu.SemaphoreType.DMA,
        ],
    )
    def sc_add(x_hbm, pe_hbm, out_hbm, xbuf, pebuf, obuf,
               xs0, xs1, ps0, ps1, os0, os1):
        wid = lax.axis_index("s") * _NC + lax.axis_index("c")
        pe0 = wid * seg
        xsem = (xs0, xs1)
        psem = (ps0, ps1)
        osem = (os0, os1)

        def x_row(ci, b):
            return b * L + pe0 + ci * _CHR

        def load_descs(ci, par):
            xs = [pltpu.make_async_copy(x_hbm.at[pl.ds(x_row(ci, b), _CHR)],
                                        xbuf.at[par, b], xsem[par])
                  for b in range(B)]
            ps = pltpu.make_async_copy(pe_hbm.at[pl.ds(pe0 + ci * _CHR, _CHR)],
                                       pebuf.at[par], psem[par])
            return xs + [ps]

        def store_descs(ci, par):
            return [pltpu.make_async_copy(obuf.at[par, b],
                                          out_hbm.at[pl.ds(x_row(ci, b), _CHR)],
                                          osem[par])
                    for b in range(B)]

        for par in range(2):
            for cp in load_descs(par, par):
                cp.start()

        def body(h, _):
            for par in range(2):
                ci = 2 * h + par
                for cp in load_descs(ci, par):
                    cp.wait()

                @pl.when(h >= 1)
                def _():
                    for cp in store_descs(ci - 2, par):
                        cp.wait()

                @plsc.parallel_loop(0, _CHR)
                def _(i):
                    for j in range(D // _LANES):
                        s = pl.ds(j * _LANES, _LANES)
                        pv = pebuf[par, i, s]
                        for b in range(B):
                            obuf[par, b, i, s] = xbuf[par, b, i, s] + pv

                for cp in store_descs(ci, par):
                    cp.start()

                @pl.when(h < n_half - 1)
                def _():
                    for cp in load_descs(ci + 2, par):
                        cp.start()
            return 0

        lax.fori_loop(0, n_half, body, 0)

        for par in range(2):
            for cp in store_descs(n_chunks - 2 + par, par):
                cp.wait()

    out = sc_add(xf, pef)
    return out.reshape(B, L, D)
